# baseline (device time: 90436 ns/iter reference)
import jax
import jax.numpy as jnp
from jax import lax
from jax.experimental import pallas as pl
from jax.experimental.pallas import tpu as pltpu

N_DEV = 16
B, Sq, Skv = 2, 256, 256
HQ, DH = 4, 64
DM = 512
HD = HQ * DH
NCHUNK = N_DEV
CROWS = (B * Sq) // NCHUNK
BLK = 64


def kernel(x, Wq, K_ext, V_ext, Wo):
    my = lax.axis_index("i")
    wq_loc = lax.dynamic_slice_in_dim(Wq, my * HD, HD, axis=1)
    wo_loc = lax.dynamic_slice_in_dim(Wo, my * HD, HD, axis=0)

    def body(x_ref, wq_ref, k_ref, v_ref, wo_ref, out_ref,
             acc_ref, rs_ref, rs_send, rs_recv, ag_send, ag_recv):
        my_pos = lax.axis_index("i")
        left = lax.rem(my_pos + N_DEV - 1, N_DEV)
        right = lax.rem(my_pos + 1, N_DEV)

        barrier = pltpu.get_barrier_semaphore()
        for nbr in (left, right):
            pl.semaphore_signal(barrier, inc=1, device_id=(nbr,),
                                device_id_type=pl.DeviceIdType.MESH)
        pl.semaphore_wait(barrier, 2)

        xf = x_ref[:].reshape(B * Sq, DM)
        q = jnp.dot(xf, wq_ref[:], preferred_element_type=jnp.float32)

        qb = lax.broadcasted_iota(jnp.int32, (Sq, Skv), 0) // BLK
        kb = lax.broadcasted_iota(jnp.int32, (Sq, Skv), 1) // BLK
        mask = kb <= qb

        for b in range(B):
            pb = jnp.zeros((Sq, DM), jnp.float32)
            for h in range(HQ):
                q_bh = q[b * Sq:(b + 1) * Sq, h * DH:(h + 1) * DH]
                k_bh = k_ref[b, :, h, :]
                v_bh = v_ref[b, :, h, :]
                s = jnp.dot(q_bh, k_bh.T,
                            preferred_element_type=jnp.float32) * 0.125
                s = jnp.where(mask, s, -1e9)
                w = jnp.exp(s - jnp.max(s, axis=-1, keepdims=True))
                w = w / jnp.sum(w, axis=-1, keepdims=True)
                ctx = jnp.dot(w, v_bh, preferred_element_type=jnp.float32)
                pb = pb + jnp.dot(ctx, wo_ref[h * DH:(h + 1) * DH, :],
                                  preferred_element_type=jnp.float32)
            nb = Sq // CROWS
            acc_ref[b * nb:(b + 1) * nb] = pb.reshape(nb, CROWS, DM)

        for s in range(N_DEV - 1):
            send_c = lax.rem(my_pos - s + N_DEV, N_DEV)
            rdma = pltpu.make_async_remote_copy(
                src_ref=acc_ref.at[send_c],
                dst_ref=rs_ref.at[s],
                send_sem=rs_send.at[s],
                recv_sem=rs_recv.at[s],
                device_id=(right,),
                device_id_type=pl.DeviceIdType.MESH,
            )
            rdma.start()
            rdma.wait()
            recv_c = lax.rem(my_pos - s - 1 + 2 * N_DEV, N_DEV)
            acc_ref[recv_c] = acc_ref[recv_c] + rs_ref[s]

        for s in range(N_DEV - 1):
            send_c = lax.rem(my_pos + 1 - s + 2 * N_DEV, N_DEV)
            rdma = pltpu.make_async_remote_copy(
                src_ref=acc_ref.at[send_c],
                dst_ref=acc_ref.at[send_c],
                send_sem=ag_send.at[s],
                recv_sem=ag_recv.at[s],
                device_id=(right,),
                device_id_type=pl.DeviceIdType.MESH,
            )
            rdma.start()
            rdma.wait()

        out_ref[:] = acc_ref[:].reshape(B, Sq, DM)

    return pl.pallas_call(
        body,
        out_shape=jax.ShapeDtypeStruct((B, Sq, DM), jnp.float32),
        in_specs=[pl.BlockSpec(memory_space=pltpu.VMEM)] * 5,
        out_specs=pl.BlockSpec(memory_space=pltpu.VMEM),
        scratch_shapes=[
            pltpu.VMEM((NCHUNK, CROWS, DM), jnp.float32),
            pltpu.VMEM((N_DEV - 1, CROWS, DM), jnp.float32),
            pltpu.SemaphoreType.DMA((N_DEV - 1,)),
            pltpu.SemaphoreType.DMA((N_DEV - 1,)),
            pltpu.SemaphoreType.DMA((N_DEV - 1,)),
            pltpu.SemaphoreType.DMA((N_DEV - 1,)),
        ],
        compiler_params=pltpu.CompilerParams(collective_id=0),
    )(x, wq_loc, K_ext, V_ext, wo_loc)


# device time: 51048 ns/iter; 1.7716x vs baseline; 1.7716x over previous
import jax
import jax.numpy as jnp
from jax import lax
from jax.experimental import pallas as pl
from jax.experimental.pallas import tpu as pltpu

N_DEV = 16
B, Sq, Skv = 2, 256, 256
HQ, DH = 4, 64
DM = 512
HD = HQ * DH
NCHUNK = N_DEV
CROWS = (B * Sq) // NCHUNK
BLK = 64

V_ORDER = (1, 4, 2, 8)
HALVES = (8, 4, 2, 1)
STAGE_OFF = (0, 8, 12, 14)

_SHIFTS = tuple(v.bit_length() - 1 for v in V_ORDER)
PERM = tuple(
    sum(((c >> _SHIFTS[k]) & 1) << (3 - k) for k in range(4)) for c in range(16)
)


def kernel(x, Wq, K_ext, V_ext, Wo):
    my = lax.axis_index("i")
    wq_loc = lax.dynamic_slice_in_dim(Wq, my * HD, HD, axis=1)
    wo_loc = lax.dynamic_slice_in_dim(Wo, my * HD, HD, axis=0)

    def body(x_ref, wq_ref, k_ref, v_ref, wo_ref, out_ref,
             acc_ref, rs_ref, rs_send, rs_recv, ag_send, ag_recv):
        my_pos = lax.axis_index("i")
        partners = [my_pos ^ v for v in V_ORDER]
        tbits = [(my_pos >> sh) & 1 for sh in _SHIFTS]

        barrier = pltpu.get_barrier_semaphore()
        for p in partners:
            pl.semaphore_signal(barrier, inc=1, device_id=(p,),
                                device_id_type=pl.DeviceIdType.MESH)
        pl.semaphore_wait(barrier, 4)

        xf = x_ref[:].reshape(B * Sq, DM)
        q = jnp.dot(xf, wq_ref[:], preferred_element_type=jnp.float32)

        qb = lax.broadcasted_iota(jnp.int32, (Sq, Skv), 0) // BLK
        kb = lax.broadcasted_iota(jnp.int32, (Sq, Skv), 1) // BLK
        mask = kb <= qb

        for b in range(B):
            pb = jnp.zeros((Sq, DM), jnp.float32)
            for h in range(HQ):
                q_bh = q[b * Sq:(b + 1) * Sq, h * DH:(h + 1) * DH]
                k_bh = k_ref[b, :, h, :]
                v_bh = v_ref[b, :, h, :]
                s = jnp.dot(q_bh, k_bh.T,
                            preferred_element_type=jnp.float32) * 0.125
                s = jnp.where(mask, s, -1e9)
                w = jnp.exp(s - jnp.max(s, axis=-1, keepdims=True))
                w = w / jnp.sum(w, axis=-1, keepdims=True)
                ctx = jnp.dot(w, v_bh, preferred_element_type=jnp.float32)
                pb = pb + jnp.dot(ctx, wo_ref[h * DH:(h + 1) * DH, :],
                                  preferred_element_type=jnp.float32)
            pb_r = pb.reshape(Sq // CROWS, CROWS, DM)
            for j in range(Sq // CROWS):
                acc_ref[PERM[b * (Sq // CROWS) + j]] = pb_r[j]

        base = my_pos * 0
        for k in range(4):
            half = HALVES[k]
            t = tbits[k]
            send_start = base + (1 - t) * half
            keep_start = base + t * half
            rdma = pltpu.make_async_remote_copy(
                src_ref=acc_ref.at[pl.ds(send_start, half)],
                dst_ref=rs_ref.at[pl.ds(STAGE_OFF[k], half)],
                send_sem=rs_send.at[k],
                recv_sem=rs_recv.at[k],
                device_id=(partners[k],),
                device_id_type=pl.DeviceIdType.MESH,
            )
            rdma.start()
            rdma.wait()
            acc_ref[pl.ds(keep_start, half)] = (
                acc_ref[pl.ds(keep_start, half)]
                + rs_ref[pl.ds(STAGE_OFF[k], half)]
            )
            base = keep_start

        for k in (3, 2, 1, 0):
            size = HALVES[k]
            t = tbits[k]
            rdma = pltpu.make_async_remote_copy(
                src_ref=acc_ref.at[pl.ds(base, size)],
                dst_ref=acc_ref.at[pl.ds(base, size)],
                send_sem=ag_send.at[k],
                recv_sem=ag_recv.at[k],
                device_id=(partners[k],),
                device_id_type=pl.DeviceIdType.MESH,
            )
            rdma.start()
            rdma.wait()
            base = base - t * size

        for c in range(NCHUNK):
            b, j = divmod(c, Sq // CROWS)
            out_ref[b, j * CROWS:(j + 1) * CROWS, :] = acc_ref[PERM[c]]

    return pl.pallas_call(
        body,
        out_shape=jax.ShapeDtypeStruct((B, Sq, DM), jnp.float32),
        in_specs=[pl.BlockSpec(memory_space=pltpu.VMEM)] * 5,
        out_specs=pl.BlockSpec(memory_space=pltpu.VMEM),
        scratch_shapes=[
            pltpu.VMEM((NCHUNK, CROWS, DM), jnp.float32),
            pltpu.VMEM((15, CROWS, DM), jnp.float32),
            pltpu.SemaphoreType.DMA((4,)),
            pltpu.SemaphoreType.DMA((4,)),
            pltpu.SemaphoreType.DMA((4,)),
            pltpu.SemaphoreType.DMA((4,)),
        ],
        compiler_params=pltpu.CompilerParams(collective_id=0),
    )(x, wq_loc, K_ext, V_ext, wo_loc)


# device time: 40442 ns/iter; 2.2362x vs baseline; 1.2623x over previous
import jax
import jax.numpy as jnp
from jax import lax
from jax.experimental import pallas as pl
from jax.experimental.pallas import tpu as pltpu

N_DEV = 16
B, Sq, Skv = 2, 256, 256
HQ, DH = 4, 64
DM = 512
HD = HQ * DH
NCHUNK = N_DEV
CROWS = (B * Sq) // NCHUNK
BLK = 64

V_ORDER = (1, 4, 2, 8)
HALVES = (8, 4, 2, 1)
STAGE_OFF = (0, 8, 12, 14)

_SHIFTS = tuple(v.bit_length() - 1 for v in V_ORDER)
PERM = tuple(
    sum(((c >> _SHIFTS[k]) & 1) << (3 - k) for k in range(4)) for c in range(16)
)


def kernel(x, Wq, K_ext, V_ext, Wo):
    my = lax.axis_index("i")
    wq_loc = lax.dynamic_slice_in_dim(Wq, my * HD, HD, axis=1)
    wo_loc = lax.dynamic_slice_in_dim(Wo, my * HD, HD, axis=0)

    def body(x_ref, wq_ref, k_ref, v_ref, wo_ref, out_ref,
             acc_ref, rs_ref, rs_send, rs_recv, ag_send, ag_recv):
        my_pos = lax.axis_index("i")
        partners = [my_pos ^ v for v in V_ORDER]
        tbits = [(my_pos >> sh) & 1 for sh in _SHIFTS]

        barrier = pltpu.get_barrier_semaphore()
        for p in partners:
            pl.semaphore_signal(barrier, inc=1, device_id=(p,),
                                device_id_type=pl.DeviceIdType.MESH)
        pl.semaphore_wait(barrier, 4)

        xf = x_ref[:].reshape(B * Sq, DM)
        q = jnp.dot(xf, wq_ref[:], preferred_element_type=jnp.float32)

        qb = lax.broadcasted_iota(jnp.int32, (Sq, Skv), 0) // BLK
        kb = lax.broadcasted_iota(jnp.int32, (Sq, Skv), 1) // BLK
        mask = kb <= qb

        for b in range(B):
            pb = jnp.zeros((Sq, DM), jnp.float32)
            for h in range(HQ):
                q_bh = q[b * Sq:(b + 1) * Sq, h * DH:(h + 1) * DH]
                k_bh = k_ref[b, :, h, :]
                v_bh = v_ref[b, :, h, :]
                s = jnp.dot(q_bh, k_bh.T,
                            preferred_element_type=jnp.float32) * 0.125
                s = jnp.where(mask, s, -1e9)
                w = jnp.exp(s - jnp.max(s, axis=-1, keepdims=True))
                w = w / jnp.sum(w, axis=-1, keepdims=True)
                ctx = jnp.dot(w, v_bh, preferred_element_type=jnp.float32)
                pb = pb + jnp.dot(ctx, wo_ref[h * DH:(h + 1) * DH, :],
                                  preferred_element_type=jnp.float32)
            pb_r = pb.astype(jnp.bfloat16).reshape(Sq // CROWS, CROWS, DM)
            for j in range(Sq // CROWS):
                acc_ref[PERM[b * (Sq // CROWS) + j]] = pb_r[j]

        base = my_pos * 0
        for k in range(4):
            half = HALVES[k]
            t = tbits[k]
            send_start = base + (1 - t) * half
            keep_start = base + t * half
            rdma = pltpu.make_async_remote_copy(
                src_ref=acc_ref.at[pl.ds(send_start, half)],
                dst_ref=rs_ref.at[pl.ds(STAGE_OFF[k], half)],
                send_sem=rs_send.at[k],
                recv_sem=rs_recv.at[k],
                device_id=(partners[k],),
                device_id_type=pl.DeviceIdType.MESH,
            )
            rdma.start()
            rdma.wait()
            acc_ref[pl.ds(keep_start, half)] = (
                acc_ref[pl.ds(keep_start, half)].astype(jnp.float32)
                + rs_ref[pl.ds(STAGE_OFF[k], half)].astype(jnp.float32)
            ).astype(jnp.bfloat16)
            base = keep_start

        for k in (3, 2, 1, 0):
            size = HALVES[k]
            t = tbits[k]
            rdma = pltpu.make_async_remote_copy(
                src_ref=acc_ref.at[pl.ds(base, size)],
                dst_ref=acc_ref.at[pl.ds(base, size)],
                send_sem=ag_send.at[k],
                recv_sem=ag_recv.at[k],
                device_id=(partners[k],),
                device_id_type=pl.DeviceIdType.MESH,
            )
            rdma.start()
            rdma.wait()
            base = base - t * size

        for c in range(NCHUNK):
            b, j = divmod(c, Sq // CROWS)
            out_ref[b, j * CROWS:(j + 1) * CROWS, :] = (
                acc_ref[PERM[c]].astype(jnp.float32))

    return pl.pallas_call(
        body,
        out_shape=jax.ShapeDtypeStruct((B, Sq, DM), jnp.float32),
        in_specs=[pl.BlockSpec(memory_space=pltpu.VMEM)] * 5,
        out_specs=pl.BlockSpec(memory_space=pltpu.VMEM),
        scratch_shapes=[
            pltpu.VMEM((NCHUNK, CROWS, DM), jnp.bfloat16),
            pltpu.VMEM((15, CROWS, DM), jnp.bfloat16),
            pltpu.SemaphoreType.DMA((4,)),
            pltpu.SemaphoreType.DMA((4,)),
            pltpu.SemaphoreType.DMA((4,)),
            pltpu.SemaphoreType.DMA((4,)),
        ],
        compiler_params=pltpu.CompilerParams(collective_id=0),
    )(x, wq_loc, K_ext, V_ext, wo_loc)


# device time: 30661 ns/iter; 2.9495x vs baseline; 1.3190x over previous
import jax
import jax.numpy as jnp
from jax import lax
from jax.experimental import pallas as pl
from jax.experimental.pallas import tpu as pltpu

N_DEV = 16
B, Sq, Skv = 2, 256, 256
HQ, DH = 4, 64
DM = 512
HD = HQ * DH
NCHUNK = N_DEV
CROWS = (B * Sq) // NCHUNK
CPB = Sq // CROWS
BLK = 64


def kernel(x, Wq, K_ext, V_ext, Wo):
    my = lax.axis_index("i")
    wq_loc = lax.dynamic_slice_in_dim(Wq, my * HD, HD, axis=1)
    wo_loc = lax.dynamic_slice_in_dim(Wo, my * HD, HD, axis=0)

    def body(x_ref, wq_ref, k_ref, v_ref, wo_ref, out_ref,
             acc_ref, rs_ref, rs_send, rs_recv, ag_send, ag_recv):
        my_pos = lax.axis_index("i")

        barrier = pltpu.get_barrier_semaphore()
        for d in range(N_DEV):
            pl.semaphore_signal(barrier, inc=1, device_id=(d,),
                                device_id_type=pl.DeviceIdType.MESH)

        qb = lax.broadcasted_iota(jnp.int32, (Sq, Skv), 0) // BLK
        kb = lax.broadcasted_iota(jnp.int32, (Sq, Skv), 1) // BLK
        mask = kb <= qb

        def compute_batch(b):
            xf = x_ref[b]
            q = jnp.dot(xf, wq_ref[:], preferred_element_type=jnp.float32)
            pb = jnp.zeros((Sq, DM), jnp.float32)
            for h in range(HQ):
                q_bh = q[:, h * DH:(h + 1) * DH]
                k_bh = k_ref[b, :, h, :]
                v_bh = v_ref[b, :, h, :]
                s = jnp.dot(q_bh, k_bh.T,
                            preferred_element_type=jnp.float32) * 0.125
                s = jnp.where(mask, s, -1e9)
                w = jnp.exp(s - jnp.max(s, axis=-1, keepdims=True))
                w = w / jnp.sum(w, axis=-1, keepdims=True)
                ctx = jnp.dot(w, v_bh, preferred_element_type=jnp.float32)
                pb = pb + jnp.dot(ctx, wo_ref[h * DH:(h + 1) * DH, :],
                                  preferred_element_type=jnp.float32)
            pb_r = pb.astype(jnp.bfloat16).reshape(CPB, CROWS, DM)
            for j in range(CPB):
                acc_ref[b * CPB + j] = pb_r[j]

        rs_descs = []

        def send_chunks(c_lo, c_hi):
            for c in range(c_lo, c_hi):
                rdma = pltpu.make_async_remote_copy(
                    src_ref=acc_ref.at[c],
                    dst_ref=rs_ref.at[my_pos],
                    send_sem=rs_send.at[c],
                    recv_sem=rs_recv.at[my_pos],
                    device_id=(c,),
                    device_id_type=pl.DeviceIdType.MESH,
                )
                rdma.start()
                rs_descs.append(rdma)

        compute_batch(0)
        pl.semaphore_wait(barrier, N_DEV)
        send_chunks(0, CPB)
        compute_batch(1)
        send_chunks(CPB, NCHUNK)

        for s in range(N_DEV):
            recv = pltpu.make_async_remote_copy(
                src_ref=rs_ref.at[s], dst_ref=rs_ref.at[s],
                send_sem=rs_send.at[s], recv_sem=rs_recv.at[s],
                device_id=(s,), device_id_type=pl.DeviceIdType.MESH,
            )
            recv.wait_recv()
        total = jnp.sum(rs_ref[:].astype(jnp.float32), axis=0)
        acc_ref[my_pos] = total.astype(jnp.bfloat16)

        ag_descs = []
        for d in range(N_DEV):
            @pl.when(my_pos != d)
            def _():
                rdma = pltpu.make_async_remote_copy(
                    src_ref=acc_ref.at[my_pos],
                    dst_ref=acc_ref.at[my_pos],
                    send_sem=ag_send.at[d],
                    recv_sem=ag_recv.at[my_pos],
                    device_id=(d,),
                    device_id_type=pl.DeviceIdType.MESH,
                )
                rdma.start()

        for s in range(N_DEV):
            @pl.when(my_pos != s)
            def _():
                recv = pltpu.make_async_remote_copy(
                    src_ref=acc_ref.at[s], dst_ref=acc_ref.at[s],
                    send_sem=ag_send.at[s], recv_sem=ag_recv.at[s],
                    device_id=(s,), device_id_type=pl.DeviceIdType.MESH,
                )
                recv.wait_recv()

        for rdma in rs_descs:
            rdma.wait_send()
        for d in range(N_DEV):
            @pl.when(my_pos != d)
            def _():
                send = pltpu.make_async_remote_copy(
                    src_ref=acc_ref.at[my_pos], dst_ref=acc_ref.at[my_pos],
                    send_sem=ag_send.at[d], recv_sem=ag_recv.at[my_pos],
                    device_id=(d,), device_id_type=pl.DeviceIdType.MESH,
                )
                send.wait_send()

        for c in range(NCHUNK):
            b, j = divmod(c, CPB)
            out_ref[b, j * CROWS:(j + 1) * CROWS, :] = (
                acc_ref[c].astype(jnp.float32))

    return pl.pallas_call(
        body,
        out_shape=jax.ShapeDtypeStruct((B, Sq, DM), jnp.float32),
        in_specs=[pl.BlockSpec(memory_space=pltpu.VMEM)] * 5,
        out_specs=pl.BlockSpec(memory_space=pltpu.VMEM),
        scratch_shapes=[
            pltpu.VMEM((NCHUNK, CROWS, DM), jnp.bfloat16),
            pltpu.VMEM((N_DEV, CROWS, DM), jnp.bfloat16),
            pltpu.SemaphoreType.DMA((N_DEV,)),
            pltpu.SemaphoreType.DMA((N_DEV,)),
            pltpu.SemaphoreType.DMA((N_DEV,)),
            pltpu.SemaphoreType.DMA((N_DEV,)),
        ],
        compiler_params=pltpu.CompilerParams(collective_id=0),
    )(x, wq_loc, K_ext, V_ext, wo_loc)
